# Initial kernel scaffold; baseline (speedup 1.0000x reference)
import jax
import jax.numpy as jnp
from jax.experimental import pallas as pl


def kernel(x, edge_index, user_indices, item_indices, emb, W1, as1, ad1, b1, W2, as2, ad2, b2, fw1, fb1, fg, fbe, fw2, fb2, pw1, pb1, pg, pbe, pw2, pb2, pw3, pb3):
    raise NotImplementedError("write your pallas kernel here")


# speedup vs baseline: 1.0928x; 1.0928x over previous
import functools

import jax
import jax.numpy as jnp
from jax.experimental import pallas as pl
from jax.experimental.pallas import tpu as pltpu


def _gat_layer(h, src, dst, W, att_src, att_dst, bias, heads, out_ch, concat):
    N = h.shape[0]
    xh = (h @ W).reshape(N, heads, out_ch)
    a_src = (xh * att_src[None]).sum(-1)
    a_dst = (xh * att_dst[None]).sum(-1)
    alpha = jax.nn.leaky_relu(a_src[src] + a_dst[dst], 0.2)
    ex = jnp.exp(alpha)
    denom = jax.ops.segment_sum(ex, dst, num_segments=N)
    att = ex / (denom[dst] + 1e-16)
    out = jax.ops.segment_sum(xh[src] * att[:, :, None], dst, num_segments=N)
    if concat:
        out = out.reshape(N, heads * out_ch)
    else:
        out = out.mean(axis=1)
    return out + bias


_BN_SCALE = 1.0 / (1.0 + 1e-5) ** 0.5


def _head_body(ue_ref, ie_ref, fw1_ref, fb1_ref, fg_ref, fbe_ref, fw2_ref,
               fb2_ref, pw1_ref, pb1_ref, pg_ref, pbe_ref, pw2_ref, pb2_ref,
               pw3_ref, pb3_ref, out_ref):
    ue = ue_ref[...]
    ie = ie_ref[...]
    fw1s = fw1_ref[...]
    fb1 = fb1_ref[...]
    fscale = fg_ref[...] * _BN_SCALE
    fbe = fbe_ref[...]
    fw2 = fw2_ref[...]
    fb2 = fb2_ref[...]

    def fusion(e):
        z = jnp.dot(e, fw1s, preferred_element_type=jnp.float32) + fb1
        z = jnp.maximum(z * fscale + fbe, 0.0)
        return jnp.dot(z, fw2, preferred_element_type=jnp.float32) + fb2

    uef = fusion(ue)
    ief = fusion(ie)
    pw1 = pw1_ref[...]
    z = (jnp.dot(uef, pw1[:64], preferred_element_type=jnp.float32)
         + jnp.dot(ief, pw1[64:], preferred_element_type=jnp.float32)
         + pb1_ref[...])
    z = jnp.maximum(z * (pg_ref[...] * _BN_SCALE) + pbe_ref[...], 0.0)
    z = jnp.maximum(jnp.dot(z, pw2_ref[...], preferred_element_type=jnp.float32)
                    + pb2_ref[...], 0.0)
    out_ref[...] = (jnp.dot(z, pw3_ref[...], preferred_element_type=jnp.float32)
                    + pb3_ref[...])


def _mlp_head(ue, ie, fw1, fb1, fg, fbe, fw2, fb2,
              pw1, pb1, pg, pbe, pw2, pb2, pw3, pb3):
    B = ue.shape[0]
    BT = 1024
    fw1s = fw1[:64] + fw1[64:]
    grid = (B // BT,)
    bspec_b = pl.BlockSpec((BT, 64), lambda i: (i, 0))
    wspec = lambda s: pl.BlockSpec(s, lambda i: tuple(0 for _ in s))
    out = pl.pallas_call(
        _head_body,
        grid=grid,
        in_specs=[bspec_b, bspec_b,
                  wspec(fw1s.shape), wspec(fb1.shape), wspec(fg.shape),
                  wspec(fbe.shape), wspec(fw2.shape), wspec(fb2.shape),
                  wspec(pw1.shape), wspec(pb1.shape), wspec(pg.shape),
                  wspec(pbe.shape), wspec(pw2.shape), wspec(pb2.shape),
                  wspec(pw3.shape), wspec(pb3.shape)],
        out_specs=pl.BlockSpec((BT, 1), lambda i: (i, 0)),
        out_shape=jax.ShapeDtypeStruct((B, 1), jnp.float32),
    )(ue, ie, fw1s, fb1, fg, fbe, fw2, fb2,
      pw1, pb1, pg, pbe, pw2, pb2, pw3, pb3)
    return out[:, 0]


def kernel(x, edge_index, user_indices, item_indices, emb, W1, as1, ad1, b1,
           W2, as2, ad2, b2, fw1, fb1, fg, fbe, fw2, fb2,
           pw1, pb1, pg, pbe, pw2, pb2, pw3, pb3):
    N = emb.shape[0]
    loops = jnp.arange(N, dtype=edge_index.dtype)
    src = jnp.concatenate([edge_index[0], loops])
    dst = jnp.concatenate([edge_index[1], loops])
    h = emb[x]
    h = _gat_layer(h, src, dst, W1, as1, ad1, b1, 4, 32, True)
    h = jax.nn.elu(h)
    h = _gat_layer(h, src, dst, W2, as2, ad2, b2, 1, 64, False)
    ue = h[user_indices]
    ie = h[item_indices]
    return _mlp_head(ue, ie, fw1, fb1, fg, fbe, fw2, fb2,
                     pw1, pb1, pg, pbe, pw2, pb2, pw3, pb3)


# speedup vs baseline: 12.7691x; 11.6846x over previous
import jax
import jax.numpy as jnp
from jax import lax
from jax.experimental import pallas as pl
from jax.experimental.pallas import tpu as pltpu
from jax.experimental.pallas import tpu_sc as plsc

N = 50000
NP = 50048
EMB = 64
HID = 128
NC, NS = 2, 16
NW = NC * NS
SUB = 128
KS = 4
BE = SUB * KS
NBLK = 52
TPT = NBLK * KS
NR = NW * TPT
EPAD = NR * SUB
NPT16 = NP // NS

_mesh = plsc.VectorSubcoreMesh(core_axis_name="c", subcore_axis_name="s")
_sc_params = pltpu.CompilerParams(use_tc_tiling_on_sc=False,
                                  needs_layout_passes=False)


def _wid():
    return lax.axis_index("c") * NS + lax.axis_index("s")


def _sc_a_body(src2d, dst2d, ast, adt, e_out, dpart, *rest):
    idxs = rest[0:KS]
    idxd = rest[KS:2 * KS]
    srows, drows, ebuf, zbuf, sem, accum = rest[2 * KS:]
    c = lax.axis_index("c")
    s = lax.axis_index("s")
    wid = c * NS + s
    zero16 = jnp.zeros((16,), jnp.float32)

    @pl.loop(0, NPT16)
    def _(i):
        zbuf[i, :] = zero16

    pltpu.sync_copy(zbuf, accum.at[pl.ds(s * NPT16, NPT16)])
    plsc.subcore_barrier()

    r0 = wid * TPT

    @pl.loop(0, NBLK)
    def _(b):
        r = r0 + b * KS
        ebase = r * SUB
        for j in range(KS):
            pltpu.sync_copy(src2d.at[r + j], idxs[j])
            pltpu.sync_copy(dst2d.at[r + j], idxd[j])
            pltpu.async_copy(ast.at[idxs[j]],
                             srows.at[pl.ds(j * SUB, SUB)], sem).wait()
            pltpu.async_copy(adt.at[idxd[j]],
                             drows.at[pl.ds(j * SUB, SUB)], sem).wait()

        @pl.loop(0, BE)
        def _(i):
            xv = srows[i, :] + drows[i, :]
            xv = jnp.maximum(xv, xv * 0.2)
            ebuf[i, :] = jnp.exp(xv)

        pltpu.sync_copy(ebuf, e_out.at[pl.ds(ebase, BE)])
        for j in range(KS):
            pltpu.sync_copy(ebuf.at[pl.ds(j * SUB, SUB)],
                            accum.at[idxd[j]], add=True)

    plsc.subcore_barrier()
    for cc in range(NC):
        @pl.when(c == cc)
        def _():
            pltpu.sync_copy(accum.at[pl.ds(s * NPT16, NPT16)],
                            dpart.at[cc, pl.ds(s * NPT16, NPT16)])


_sc_a = pl.kernel(
    _sc_a_body,
    out_type=(jax.ShapeDtypeStruct((EPAD, 16), jnp.float32),
              jax.ShapeDtypeStruct((NC, NP, 16), jnp.float32)),
    mesh=_mesh,
    compiler_params=_sc_params,
    scratch_types=[pltpu.VMEM((SUB,), jnp.int32)] * (2 * KS) + [
        pltpu.VMEM((BE, 16), jnp.float32),
        pltpu.VMEM((BE, 16), jnp.float32),
        pltpu.VMEM((BE, 16), jnp.float32),
        pltpu.VMEM((NPT16, 16), jnp.float32),
        pltpu.SemaphoreType.DMA,
        pltpu.VMEM_SHARED((NP, 16), jnp.float32),
    ],
)


def _sc_a2_body(dst2d, e_in, dinvt, att_t, *rest):
    idxd = rest[0:KS]
    e2d, drows, attflat, attcol, sem = rest[KS:]
    wid = _wid()
    iota = lax.iota(jnp.int32, 16)
    r0 = wid * TPT

    @pl.loop(0, NBLK)
    def _(b):
        r = r0 + b * KS
        ebase = r * SUB
        pltpu.sync_copy(e_in.at[pl.ds(ebase, BE)], e2d)
        for j in range(KS):
            pltpu.sync_copy(dst2d.at[r + j], idxd[j])
            pltpu.async_copy(dinvt.at[idxd[j]],
                             drows.at[pl.ds(j * SUB, SUB)], sem).wait()

        @pl.loop(0, BE)
        def _(i):
            attflat[pl.ds(i * 16, 16)] = e2d[i, :] * drows[i, :]

        for h in range(4):
            @pl.loop(0, BE // 16)
            def _(g):
                idx = iota * 16 + (g * 256 + h)
                attcol[h, pl.ds(g * 16, 16)] = plsc.load_gather(attflat, [idx])
            pltpu.sync_copy(attcol.at[h], att_t.at[h, pl.ds(ebase, BE)])


_sc_a2 = pl.kernel(
    _sc_a2_body,
    out_type=jax.ShapeDtypeStruct((4, EPAD), jnp.float32),
    mesh=_mesh,
    compiler_params=_sc_params,
    scratch_types=[pltpu.VMEM((SUB,), jnp.int32)] * KS + [
        pltpu.VMEM((BE, 16), jnp.float32),
        pltpu.VMEM((BE, 16), jnp.float32),
        pltpu.VMEM((BE * 16,), jnp.float32),
        pltpu.VMEM((4, BE), jnp.float32),
        pltpu.SemaphoreType.DMA,
    ],
)


def _build_sc_b(upc, att_rows):
    n_units = NC * upc
    nrows_t = NPT16
    zr = nrows_t // 8

    def body(*args):
        src2d, dst2d, att_t = args[0:3]
        tables = args[3:3 + n_units]
        out_hbm = args[3 + n_units]
        rest = args[4 + n_units:]
        idxs = rest[0:KS]
        idxd = rest[KS:2 * KS]
        attbuf, xrows, msg, zbuf, sem, accum = rest[2 * KS:]
        c = lax.axis_index("c")
        s = lax.axis_index("s")
        wid = c * NS + s
        zero16 = jnp.zeros((16,), jnp.float32)

        @pl.loop(0, zr)
        def _(i):
            zbuf[i, :] = zero16

        tpt_b = NR // NS
        nblk_b = tpt_b // KS
        r0 = s * tpt_b
        for cc in range(NC):
            @pl.when(c == cc)
            def _():
                for jj in range(upc):
                    u = cc * upc + jj
                    table = tables[u]
                    arow = att_rows[u]
                    for q in range(8):
                        pltpu.sync_copy(
                            zbuf, accum.at[pl.ds(s * nrows_t + q * zr, zr)])
                    plsc.subcore_barrier()

                    @pl.loop(0, nblk_b)
                    def _(b):
                        r = r0 + b * KS
                        ebase = r * SUB
                        pltpu.sync_copy(att_t.at[arow, pl.ds(ebase, BE)],
                                        attbuf)
                        for j in range(KS):
                            pltpu.sync_copy(src2d.at[r + j], idxs[j])
                            pltpu.sync_copy(dst2d.at[r + j], idxd[j])
                            pltpu.async_copy(table.at[idxs[j]],
                                             xrows.at[pl.ds(j * SUB, SUB)],
                                             sem).wait()

                        @pl.loop(0, BE // 4)
                        def _(i):
                            for k in range(4):
                                e = i * 4 + k
                                ab = plsc.load_gather(
                                    attbuf, [jnp.broadcast_to(e, (16,))])
                                msg[e, :] = xrows[e, :] * ab

                        for j in range(KS):
                            pltpu.sync_copy(msg.at[pl.ds(j * SUB, SUB)],
                                            accum.at[idxd[j]], add=True)

                    plsc.subcore_barrier()
                    pltpu.sync_copy(accum.at[pl.ds(s * nrows_t, nrows_t)],
                                    out_hbm.at[u, pl.ds(s * nrows_t, nrows_t)])
                    plsc.subcore_barrier()

    return pl.kernel(
        body,
        out_type=jax.ShapeDtypeStruct((n_units, NP, 16), jnp.float32),
        mesh=_mesh,
        compiler_params=_sc_params,
        scratch_types=[pltpu.VMEM((SUB,), jnp.int32)] * (2 * KS) + [
            pltpu.VMEM((BE,), jnp.float32),
            pltpu.VMEM((BE, 16), jnp.float32),
            pltpu.VMEM((BE, 16), jnp.float32),
            pltpu.VMEM((zr, 16), jnp.float32),
            pltpu.SemaphoreType.DMA,
            pltpu.VMEM_SHARED((NP, 16), jnp.float32),
        ],
    )


_sc_b_l1 = _build_sc_b(4, (0, 0, 1, 1, 2, 2, 3, 3))
_sc_b_l2 = _build_sc_b(2, (0, 0, 0, 0))


def _sc_g_body(*args):
    tabs = args[0:4]
    u2d, i2d = args[4:6]
    outs = args[6:14]
    idxb, rows, sem = args[14:]
    wid = _wid()
    for t in range(4):
        pltpu.sync_copy(u2d.at[wid], idxb)
        pltpu.async_copy(tabs[t].at[idxb], rows, sem).wait()
        pltpu.sync_copy(rows, outs[t].at[pl.ds(wid * SUB, SUB)])
        pltpu.sync_copy(i2d.at[wid], idxb)
        pltpu.async_copy(tabs[t].at[idxb], rows, sem).wait()
        pltpu.sync_copy(rows, outs[4 + t].at[pl.ds(wid * SUB, SUB)])


_sc_g = pl.kernel(
    _sc_g_body,
    out_type=tuple(jax.ShapeDtypeStruct((4096, 16), jnp.float32)
                   for _ in range(8)),
    mesh=_mesh,
    compiler_params=_sc_params,
    scratch_types=[
        pltpu.VMEM((SUB,), jnp.int32),
        pltpu.VMEM((SUB, 16), jnp.float32),
        pltpu.SemaphoreType.DMA,
    ],
)


def _tc0_body(emb_ref, w1_ref, as1_ref, ad1_ref, *out_refs):
    xh = jnp.dot(emb_ref[...], w1_ref[...], preferred_element_type=jnp.float32)
    as1 = as1_ref[...]
    ad1 = ad1_ref[...]
    for u in range(8):
        out_refs[u][...] = xh[:, 16 * u:16 * u + 16]
    a_s, a_d = [], []
    for h in range(4):
        seg = xh[:, 32 * h:32 * h + 32]
        a_s.append(jnp.sum(seg * as1[h][None, :], axis=1, keepdims=True))
        a_d.append(jnp.sum(seg * ad1[h][None, :], axis=1, keepdims=True))
    z = jnp.zeros((xh.shape[0], 12), jnp.float32)
    out_refs[8][...] = jnp.concatenate(a_s + [z], axis=1)
    out_refs[9][...] = jnp.concatenate(a_d + [z], axis=1)


def _tc0(emb_pad, W1, as1, ad1):
    BN = 128
    full = lambda a: pl.BlockSpec(a.shape, lambda i: tuple(0 for _ in a.shape))
    bs16 = pl.BlockSpec((BN, 16), lambda i: (i, 0))
    return pl.pallas_call(
        _tc0_body,
        grid=(NP // BN,),
        in_specs=[pl.BlockSpec((BN, EMB), lambda i: (i, 0)),
                  full(W1), full(as1), full(ad1)],
        out_specs=[bs16] * 10,
        out_shape=[jax.ShapeDtypeStruct((NP, 16), jnp.float32)] * 10,
    )(emb_pad, W1, as1, ad1)


def _tc_recip_body(dp_ref, out_ref):
    out_ref[...] = 1.0 / (dp_ref[0] + dp_ref[1] + 1e-16)


def _tc_recip(dpart):
    dpv = dpart.reshape(NC, NP * 16 // 128, 128)
    R = dpv.shape[1]
    BN = 16
    out = pl.pallas_call(
        _tc_recip_body,
        grid=(R // BN,),
        in_specs=[pl.BlockSpec((NC, BN, 128), lambda i: (0, i, 0))],
        out_specs=pl.BlockSpec((BN, 128), lambda i: (i, 0)),
        out_shape=jax.ShapeDtypeStruct((R, 128), jnp.float32),
    )(dpv)
    return out.reshape(NP, 16)


def _tc2_body(*refs):
    in_refs = refs[0:8]
    b1_ref, w2_ref, as2_ref, ad2_ref = refs[8:12]
    out_refs = refs[12:]
    o = (jnp.concatenate([r[...] for r in in_refs], axis=1)
         + b1_ref[...][None, :])
    h1 = jnp.where(o > 0, o, jnp.exp(jnp.minimum(o, 0.0)) - 1.0)
    xh2 = jnp.dot(h1, w2_ref[...], preferred_element_type=jnp.float32)
    for u in range(4):
        out_refs[u][...] = xh2[:, 16 * u:16 * u + 16]
    z = jnp.zeros((xh2.shape[0], 15), jnp.float32)
    a_s = jnp.sum(xh2 * as2_ref[...][0][None, :], axis=1, keepdims=True)
    a_d = jnp.sum(xh2 * ad2_ref[...][0][None, :], axis=1, keepdims=True)
    out_refs[4][...] = jnp.concatenate([a_s, z], axis=1)
    out_refs[5][...] = jnp.concatenate([a_d, z], axis=1)


def _tc2(o_parts, b1, W2, as2, ad2):
    BN = 128
    full = lambda a: pl.BlockSpec(a.shape, lambda i: tuple(0 for _ in a.shape))
    bs16 = pl.BlockSpec((BN, 16), lambda i: (i, 0))
    return pl.pallas_call(
        _tc2_body,
        grid=(NP // BN,),
        in_specs=[bs16] * 8 + [full(b1), full(W2), full(as2), full(ad2)],
        out_specs=[bs16] * 6,
        out_shape=[jax.ShapeDtypeStruct((NP, 16), jnp.float32)] * 6,
    )(*o_parts, b1, W2, as2, ad2)


_BN_SCALE = 1.0 / (1.0 + 1e-5) ** 0.5


def _head_body(*refs):
    ue_refs = refs[0:4]
    ie_refs = refs[4:8]
    (b2_ref, fw1_ref, fb1_ref, fg_ref, fbe_ref, fw2_ref, fb2_ref, pw1_ref,
     pb1_ref, pg_ref, pbe_ref, pw2_ref, pb2_ref, pw3_ref, pb3_ref,
     out_ref) = refs[8:]
    b2 = b2_ref[...]
    ue = jnp.concatenate([r[...] for r in ue_refs], axis=1) + b2[None, :]
    ie = jnp.concatenate([r[...] for r in ie_refs], axis=1) + b2[None, :]
    fw1s = fw1_ref[...]
    fb1 = fb1_ref[...]
    fscale = fg_ref[...] * _BN_SCALE
    fbe = fbe_ref[...]
    fw2 = fw2_ref[...]
    fb2 = fb2_ref[...]

    def fusion(e):
        z = jnp.dot(e, fw1s, preferred_element_type=jnp.float32) + fb1
        z = jnp.maximum(z * fscale + fbe, 0.0)
        return jnp.dot(z, fw2, preferred_element_type=jnp.float32) + fb2

    uef = fusion(ue)
    ief = fusion(ie)
    pw1 = pw1_ref[...]
    z = (jnp.dot(uef, pw1[:64], preferred_element_type=jnp.float32)
         + jnp.dot(ief, pw1[64:], preferred_element_type=jnp.float32)
         + pb1_ref[...])
    z = jnp.maximum(z * (pg_ref[...] * _BN_SCALE) + pbe_ref[...], 0.0)
    z = jnp.maximum(jnp.dot(z, pw2_ref[...], preferred_element_type=jnp.float32)
                    + pb2_ref[...], 0.0)
    out_ref[...] = (jnp.dot(z, pw3_ref[...], preferred_element_type=jnp.float32)
                    + pb3_ref[...])


def _mlp_head(ue_parts, ie_parts, b2, fw1, fb1, fg, fbe, fw2, fb2,
              pw1, pb1, pg, pbe, pw2, pb2, pw3, pb3):
    B = ue_parts[0].shape[0]
    BT = 1024
    fw1s = fw1[:64] + fw1[64:]
    bspec = pl.BlockSpec((BT, 16), lambda i: (i, 0))
    full = lambda a: pl.BlockSpec(a.shape, lambda i: tuple(0 for _ in a.shape))
    out = pl.pallas_call(
        _head_body,
        grid=(B // BT,),
        in_specs=[bspec] * 8
                 + [full(a) for a in (b2, fw1s, fb1, fg, fbe, fw2, fb2, pw1,
                                      pb1, pg, pbe, pw2, pb2, pw3, pb3)],
        out_specs=pl.BlockSpec((BT, 1), lambda i: (i, 0)),
        out_shape=jax.ShapeDtypeStruct((B, 1), jnp.float32),
    )(*ue_parts, *ie_parts, b2, fw1s, fb1, fg, fbe, fw2, fb2,
      pw1, pb1, pg, pbe, pw2, pb2, pw3, pb3)
    return out[:, 0]


def kernel(x, edge_index, user_indices, item_indices, emb, W1, as1, ad1, b1,
           W2, as2, ad2, b2, fw1, fb1, fg, fbe, fw2, fb2,
           pw1, pb1, pg, pbe, pw2, pb2, pw3, pb3):
    E = edge_index.shape[1]
    loops = jnp.arange(N, dtype=jnp.int32)
    pad = jnp.full((EPAD - E - N,), N, jnp.int32)
    src2d = jnp.concatenate([edge_index[0], loops, pad]).reshape(NR, SUB)
    dst2d = jnp.concatenate([edge_index[1], loops, pad]).reshape(NR, SUB)
    emb_pad = jnp.pad(emb, ((0, NP - N), (0, 0)))

    tc0_out = _tc0(emb_pad, W1, as1, ad1)
    x_tabs, ast1, adt1 = tc0_out[0:8], tc0_out[8], tc0_out[9]
    e1, dpart1 = _sc_a(src2d, dst2d, ast1, adt1)
    dinv1 = _tc_recip(dpart1)
    att1 = _sc_a2(dst2d, e1, dinv1)
    outb1 = _sc_b_l1(src2d, dst2d, att1, *x_tabs)

    tc2_out = _tc2([outb1[u] for u in range(8)], b1, W2, as2, ad2)
    y_tabs, ast2, adt2 = tc2_out[0:4], tc2_out[4], tc2_out[5]
    e2, dpart2 = _sc_a(src2d, dst2d, ast2, adt2)
    dinv2 = _tc_recip(dpart2)
    att2 = _sc_a2(dst2d, e2, dinv2)
    outb2 = _sc_b_l2(src2d, dst2d, att2, *y_tabs)

    u2d = user_indices.reshape(NW, SUB)
    i2d = item_indices.reshape(NW, SUB)
    g = _sc_g(outb2[0], outb2[1], outb2[2], outb2[3], u2d, i2d)
    return _mlp_head(g[0:4], g[4:8], b2, fw1, fb1, fg, fbe, fw2, fb2,
                     pw1, pb1, pg, pbe, pw2, pb2, pw3, pb3)


# speedup vs baseline: 22.9762x; 1.7994x over previous
import jax
import jax.numpy as jnp
from jax import lax
from jax.experimental import pallas as pl
from jax.experimental.pallas import tpu as pltpu
from jax.experimental.pallas import tpu_sc as plsc

N = 50000
NP = 50048
EMB = 64
HID = 128
NC, NS = 2, 16
NW = NC * NS
SUB = 128
KS = 4
BE = SUB * KS
NBLK = 52
TPT = NBLK * KS
NR = NW * TPT
EPAD = NR * SUB
NPT16 = NP // NS

_mesh = plsc.VectorSubcoreMesh(core_axis_name="c", subcore_axis_name="s")
_sc_params = pltpu.CompilerParams(use_tc_tiling_on_sc=False,
                                  needs_layout_passes=False)


def _wid():
    return lax.axis_index("c") * NS + lax.axis_index("s")


def _sc_a_body(src2d, dst2d, ast, adt, e_out, dpart, *rest):
    idxs = rest[0:KS]
    idxd = rest[KS:2 * KS]
    srows, drows, ebuf, zbuf, sem, accum = rest[2 * KS:]
    c = lax.axis_index("c")
    s = lax.axis_index("s")
    wid = c * NS + s
    zero16 = jnp.zeros((16,), jnp.float32)

    @pl.loop(0, NPT16)
    def _(i):
        zbuf[i, :] = zero16

    pltpu.sync_copy(zbuf, accum.at[pl.ds(s * NPT16, NPT16)])
    plsc.subcore_barrier()

    r0 = wid * TPT

    @pl.loop(0, NBLK)
    def _(b):
        r = r0 + b * KS
        ebase = r * SUB
        for j in range(KS):
            pltpu.sync_copy(src2d.at[r + j], idxs[j])
            pltpu.sync_copy(dst2d.at[r + j], idxd[j])
            pltpu.async_copy(ast.at[idxs[j]],
                             srows.at[pl.ds(j * SUB, SUB)], sem).wait()
            pltpu.async_copy(adt.at[idxd[j]],
                             drows.at[pl.ds(j * SUB, SUB)], sem).wait()

        @pl.loop(0, BE)
        def _(i):
            xv = srows[i, :] + drows[i, :]
            xv = jnp.maximum(xv, xv * 0.2)
            ebuf[i, :] = jnp.exp(xv)

        pltpu.sync_copy(ebuf, e_out.at[pl.ds(ebase, BE)])
        for j in range(KS):
            pltpu.sync_copy(ebuf.at[pl.ds(j * SUB, SUB)],
                            accum.at[idxd[j]], add=True)

    plsc.subcore_barrier()
    for cc in range(NC):
        @pl.when(c == cc)
        def _():
            pltpu.sync_copy(accum.at[pl.ds(s * NPT16, NPT16)],
                            dpart.at[cc, pl.ds(s * NPT16, NPT16)])


_sc_a = pl.kernel(
    _sc_a_body,
    out_type=(jax.ShapeDtypeStruct((EPAD, 16), jnp.float32),
              jax.ShapeDtypeStruct((NC, NP, 16), jnp.float32)),
    mesh=_mesh,
    compiler_params=_sc_params,
    scratch_types=[pltpu.VMEM((SUB,), jnp.int32)] * (2 * KS) + [
        pltpu.VMEM((BE, 16), jnp.float32),
        pltpu.VMEM((BE, 16), jnp.float32),
        pltpu.VMEM((BE, 16), jnp.float32),
        pltpu.VMEM((NPT16, 16), jnp.float32),
        pltpu.SemaphoreType.DMA,
        pltpu.VMEM_SHARED((NP, 16), jnp.float32),
    ],
)


def _sc_a2_body(dst2d, e_in, dinvt, att_t, *rest):
    idxd = rest[0:KS]
    e2d, drows, attflat, attcol, sem = rest[KS:]
    wid = _wid()
    iota = lax.iota(jnp.int32, 16)
    r0 = wid * TPT

    @pl.loop(0, NBLK)
    def _(b):
        r = r0 + b * KS
        ebase = r * SUB
        pltpu.sync_copy(e_in.at[pl.ds(ebase, BE)], e2d)
        for j in range(KS):
            pltpu.sync_copy(dst2d.at[r + j], idxd[j])
            pltpu.async_copy(dinvt.at[idxd[j]],
                             drows.at[pl.ds(j * SUB, SUB)], sem).wait()

        @pl.loop(0, BE)
        def _(i):
            attflat[pl.ds(i * 16, 16)] = e2d[i, :] * drows[i, :]

        for h in range(4):
            @pl.loop(0, BE // 16)
            def _(g):
                idx = iota * 16 + (g * 256 + h)
                attcol[h, pl.ds(g * 16, 16)] = plsc.load_gather(attflat, [idx])
            pltpu.sync_copy(attcol.at[h], att_t.at[h, pl.ds(ebase, BE)])


_sc_a2 = pl.kernel(
    _sc_a2_body,
    out_type=jax.ShapeDtypeStruct((4, EPAD), jnp.float32),
    mesh=_mesh,
    compiler_params=_sc_params,
    scratch_types=[pltpu.VMEM((SUB,), jnp.int32)] * KS + [
        pltpu.VMEM((BE, 16), jnp.float32),
        pltpu.VMEM((BE, 16), jnp.float32),
        pltpu.VMEM((BE * 16,), jnp.float32),
        pltpu.VMEM((4, BE), jnp.float32),
        pltpu.SemaphoreType.DMA,
    ],
)


def _build_sc_b(upc, att_rows):
    n_units = NC * upc
    nrows_t = NPT16
    zr = nrows_t // 8
    tpt_b = NR // NS
    nblk_b = tpt_b // KS
    nb4 = nblk_b // 4

    def body(*args):
        src2d, dst2d, att_t = args[0:3]
        tables = args[3:3 + n_units]
        out_hbm = args[3 + n_units]
        rest = args[4 + n_units:]
        srci = rest[0:2]
        dsti = tuple(rest[2 + 4 * q:2 + 4 * q + KS] for q in range(4))
        o = 2 + 4 * KS
        attb = rest[o:o + 2]
        xrows = rest[o + 2:o + 4]
        msg = rest[o + 4:o + 6]
        zbuf = rest[o + 6]
        si = rest[o + 7:o + 9]
        sg = rest[o + 9:o + 11]
        ss = rest[o + 11:o + 13]
        accum = rest[o + 13]
        c = lax.axis_index("c")
        s = lax.axis_index("s")
        zero16 = jnp.zeros((16,), jnp.float32)

        @pl.loop(0, zr)
        def _(i):
            zbuf[i, :] = zero16

        r0 = s * tpt_b

        def idx_copies(b, p, dq):
            r = r0 + b * KS
            yield src2d.at[pl.ds(r, KS)], srci[p], si[p]
            for j in range(KS):
                yield dst2d.at[r + j], dsti[dq][j], si[p]

        def start_idx(b, p, dq):
            for a, d, m in idx_copies(b, p, dq):
                pltpu.async_copy(a, d, m)

        def wait_idx(b, p, dq):
            for a, d, m in idx_copies(b, p, dq):
                pltpu.make_async_copy(a, d, m).wait()

        def gather_copies(b, p, table, arow):
            r = r0 + b * KS
            yield att_t.at[arow, pl.ds(r * SUB, BE)], attb[p], sg[p]
            for j in range(KS):
                yield (table.at[srci[p].at[j]],
                       xrows[p].at[pl.ds(j * SUB, SUB)], sg[p])

        def start_gathers(b, p, table, arow):
            for a, d, m in gather_copies(b, p, table, arow):
                pltpu.async_copy(a, d, m)

        def wait_gathers(b, p, table, arow):
            for a, d, m in gather_copies(b, p, table, arow):
                pltpu.make_async_copy(a, d, m).wait()

        def start_scat(p, dq):
            for j in range(KS):
                pltpu.async_copy(msg[p].at[pl.ds(j * SUB, SUB)],
                                 accum.at[dsti[dq][j]], ss[p], add=True)

        def wait_scat(p, dq):
            for j in range(KS):
                pltpu.make_async_copy(msg[p].at[pl.ds(j * SUB, SUB)],
                                      accum.at[dsti[dq][j]], ss[p]).wait()

        def compute(p):
            @pl.loop(0, BE // 4)
            def _(i):
                for k4 in range(4):
                    e = i * 4 + k4
                    ab = plsc.load_gather(attb[p],
                                          [jnp.broadcast_to(e, (16,))])
                    msg[p][e, :] = xrows[p][e, :] * ab

        for cc in range(NC):
            @pl.when(c == cc)
            def _():
                for jj in range(upc):
                    u = cc * upc + jj
                    table = tables[u]
                    arow = att_rows[u]
                    for q in range(8):
                        pltpu.sync_copy(
                            zbuf, accum.at[pl.ds(s * nrows_t + q * zr, zr)])
                    plsc.subcore_barrier()

                    start_idx(0, 0, 0)
                    start_idx(1, 1, 1)
                    wait_idx(0, 0, 0)
                    start_gathers(0, 0, table, arow)

                    @pl.loop(0, nb4)
                    def _(t2):
                        for q in range(4):
                            b = 4 * t2 + q
                            p = q % 2

                            @pl.when(b + 1 < nblk_b)
                            def _():
                                wait_idx(b + 1, 1 - p, (q + 1) % 4)
                                start_gathers(b + 1, 1 - p, table, arow)

                            wait_gathers(b, p, table, arow)

                            if q >= 2:
                                wait_scat(p, q - 2)
                            else:
                                @pl.when(t2 > 0)
                                def _():
                                    wait_scat(p, q + 2)

                            @pl.when(b + 2 < nblk_b)
                            def _():
                                start_idx(b + 2, p, (q + 2) % 4)

                            compute(p)
                            start_scat(p, q)

                    wait_scat(0, 2)
                    wait_scat(1, 3)
                    plsc.subcore_barrier()
                    pltpu.sync_copy(accum.at[pl.ds(s * nrows_t, nrows_t)],
                                    out_hbm.at[u, pl.ds(s * nrows_t, nrows_t)])
                    plsc.subcore_barrier()

    return pl.kernel(
        body,
        out_type=jax.ShapeDtypeStruct((n_units, NP, 16), jnp.float32),
        mesh=_mesh,
        compiler_params=_sc_params,
        scratch_types=(
            [pltpu.VMEM((KS, SUB), jnp.int32)] * 2
            + [pltpu.VMEM((SUB,), jnp.int32)] * (4 * KS)
            + [pltpu.VMEM((BE,), jnp.float32)] * 2
            + [pltpu.VMEM((BE, 16), jnp.float32)] * 4
            + [pltpu.VMEM((zr, 16), jnp.float32)]
            + [pltpu.SemaphoreType.DMA] * 6
            + [pltpu.VMEM_SHARED((NP, 16), jnp.float32)]
        ),
    )


_sc_b_l1 = _build_sc_b(4, (0, 0, 1, 1, 2, 2, 3, 3))
_sc_b_l2 = _build_sc_b(2, (0, 0, 0, 0))


def _sc_g_body(*args):
    tabs = args[0:4]
    u2d, i2d = args[4:6]
    outs = args[6:14]
    idxb, rows, sem = args[14:]
    wid = _wid()
    for t in range(4):
        pltpu.sync_copy(u2d.at[wid], idxb)
        pltpu.async_copy(tabs[t].at[idxb], rows, sem).wait()
        pltpu.sync_copy(rows, outs[t].at[pl.ds(wid * SUB, SUB)])
        pltpu.sync_copy(i2d.at[wid], idxb)
        pltpu.async_copy(tabs[t].at[idxb], rows, sem).wait()
        pltpu.sync_copy(rows, outs[4 + t].at[pl.ds(wid * SUB, SUB)])


_sc_g = pl.kernel(
    _sc_g_body,
    out_type=tuple(jax.ShapeDtypeStruct((4096, 16), jnp.float32)
                   for _ in range(8)),
    mesh=_mesh,
    compiler_params=_sc_params,
    scratch_types=[
        pltpu.VMEM((SUB,), jnp.int32),
        pltpu.VMEM((SUB, 16), jnp.float32),
        pltpu.SemaphoreType.DMA,
    ],
)


def _tc0_body(emb_ref, w1_ref, as1_ref, ad1_ref, *out_refs):
    xh = jnp.dot(emb_ref[...], w1_ref[...], preferred_element_type=jnp.float32)
    as1 = as1_ref[...]
    ad1 = ad1_ref[...]
    for u in range(8):
        out_refs[u][...] = xh[:, 16 * u:16 * u + 16]
    a_s, a_d = [], []
    for h in range(4):
        seg = xh[:, 32 * h:32 * h + 32]
        a_s.append(jnp.sum(seg * as1[h][None, :], axis=1, keepdims=True))
        a_d.append(jnp.sum(seg * ad1[h][None, :], axis=1, keepdims=True))
    z = jnp.zeros((xh.shape[0], 12), jnp.float32)
    out_refs[8][...] = jnp.concatenate(a_s + [z], axis=1)
    out_refs[9][...] = jnp.concatenate(a_d + [z], axis=1)


def _tc0(emb_pad, W1, as1, ad1):
    BN = 128
    full = lambda a: pl.BlockSpec(a.shape, lambda i: tuple(0 for _ in a.shape))
    bs16 = pl.BlockSpec((BN, 16), lambda i: (i, 0))
    return pl.pallas_call(
        _tc0_body,
        grid=(NP // BN,),
        in_specs=[pl.BlockSpec((BN, EMB), lambda i: (i, 0)),
                  full(W1), full(as1), full(ad1)],
        out_specs=[bs16] * 10,
        out_shape=[jax.ShapeDtypeStruct((NP, 16), jnp.float32)] * 10,
    )(emb_pad, W1, as1, ad1)


def _tc_recip_body(dp_ref, out_ref):
    out_ref[...] = 1.0 / (dp_ref[0] + dp_ref[1] + 1e-16)


def _tc_recip(dpart):
    dpv = dpart.reshape(NC, NP * 16 // 128, 128)
    R = dpv.shape[1]
    BN = 16
    out = pl.pallas_call(
        _tc_recip_body,
        grid=(R // BN,),
        in_specs=[pl.BlockSpec((NC, BN, 128), lambda i: (0, i, 0))],
        out_specs=pl.BlockSpec((BN, 128), lambda i: (i, 0)),
        out_shape=jax.ShapeDtypeStruct((R, 128), jnp.float32),
    )(dpv)
    return out.reshape(NP, 16)


def _tc2_body(*refs):
    in_refs = refs[0:8]
    b1_ref, w2_ref, as2_ref, ad2_ref = refs[8:12]
    out_refs = refs[12:]
    o = (jnp.concatenate([r[...] for r in in_refs], axis=1)
         + b1_ref[...][None, :])
    h1 = jnp.where(o > 0, o, jnp.exp(jnp.minimum(o, 0.0)) - 1.0)
    xh2 = jnp.dot(h1, w2_ref[...], preferred_element_type=jnp.float32)
    for u in range(4):
        out_refs[u][...] = xh2[:, 16 * u:16 * u + 16]
    z = jnp.zeros((xh2.shape[0], 15), jnp.float32)
    a_s = jnp.sum(xh2 * as2_ref[...][0][None, :], axis=1, keepdims=True)
    a_d = jnp.sum(xh2 * ad2_ref[...][0][None, :], axis=1, keepdims=True)
    out_refs[4][...] = jnp.concatenate([a_s, z], axis=1)
    out_refs[5][...] = jnp.concatenate([a_d, z], axis=1)


def _tc2(o_parts, b1, W2, as2, ad2):
    BN = 128
    full = lambda a: pl.BlockSpec(a.shape, lambda i: tuple(0 for _ in a.shape))
    bs16 = pl.BlockSpec((BN, 16), lambda i: (i, 0))
    return pl.pallas_call(
        _tc2_body,
        grid=(NP // BN,),
        in_specs=[bs16] * 8 + [full(b1), full(W2), full(as2), full(ad2)],
        out_specs=[bs16] * 6,
        out_shape=[jax.ShapeDtypeStruct((NP, 16), jnp.float32)] * 6,
    )(*o_parts, b1, W2, as2, ad2)


_BN_SCALE = 1.0 / (1.0 + 1e-5) ** 0.5


def _head_body(*refs):
    ue_refs = refs[0:4]
    ie_refs = refs[4:8]
    (b2_ref, fw1_ref, fb1_ref, fg_ref, fbe_ref, fw2_ref, fb2_ref, pw1_ref,
     pb1_ref, pg_ref, pbe_ref, pw2_ref, pb2_ref, pw3_ref, pb3_ref,
     out_ref) = refs[8:]
    b2 = b2_ref[...]
    ue = jnp.concatenate([r[...] for r in ue_refs], axis=1) + b2[None, :]
    ie = jnp.concatenate([r[...] for r in ie_refs], axis=1) + b2[None, :]
    fw1s = fw1_ref[...]
    fb1 = fb1_ref[...]
    fscale = fg_ref[...] * _BN_SCALE
    fbe = fbe_ref[...]
    fw2 = fw2_ref[...]
    fb2 = fb2_ref[...]

    def fusion(e):
        z = jnp.dot(e, fw1s, preferred_element_type=jnp.float32) + fb1
        z = jnp.maximum(z * fscale + fbe, 0.0)
        return jnp.dot(z, fw2, preferred_element_type=jnp.float32) + fb2

    uef = fusion(ue)
    ief = fusion(ie)
    pw1 = pw1_ref[...]
    z = (jnp.dot(uef, pw1[:64], preferred_element_type=jnp.float32)
         + jnp.dot(ief, pw1[64:], preferred_element_type=jnp.float32)
         + pb1_ref[...])
    z = jnp.maximum(z * (pg_ref[...] * _BN_SCALE) + pbe_ref[...], 0.0)
    z = jnp.maximum(jnp.dot(z, pw2_ref[...], preferred_element_type=jnp.float32)
                    + pb2_ref[...], 0.0)
    out_ref[...] = (jnp.dot(z, pw3_ref[...], preferred_element_type=jnp.float32)
                    + pb3_ref[...])


def _mlp_head(ue_parts, ie_parts, b2, fw1, fb1, fg, fbe, fw2, fb2,
              pw1, pb1, pg, pbe, pw2, pb2, pw3, pb3):
    B = ue_parts[0].shape[0]
    BT = 1024
    fw1s = fw1[:64] + fw1[64:]
    bspec = pl.BlockSpec((BT, 16), lambda i: (i, 0))
    full = lambda a: pl.BlockSpec(a.shape, lambda i: tuple(0 for _ in a.shape))
    out = pl.pallas_call(
        _head_body,
        grid=(B // BT,),
        in_specs=[bspec] * 8
                 + [full(a) for a in (b2, fw1s, fb1, fg, fbe, fw2, fb2, pw1,
                                      pb1, pg, pbe, pw2, pb2, pw3, pb3)],
        out_specs=pl.BlockSpec((BT, 1), lambda i: (i, 0)),
        out_shape=jax.ShapeDtypeStruct((B, 1), jnp.float32),
    )(*ue_parts, *ie_parts, b2, fw1s, fb1, fg, fbe, fw2, fb2,
      pw1, pb1, pg, pbe, pw2, pb2, pw3, pb3)
    return out[:, 0]


def kernel(x, edge_index, user_indices, item_indices, emb, W1, as1, ad1, b1,
           W2, as2, ad2, b2, fw1, fb1, fg, fbe, fw2, fb2,
           pw1, pb1, pg, pbe, pw2, pb2, pw3, pb3):
    E = edge_index.shape[1]
    loops = jnp.arange(N, dtype=jnp.int32)
    pad = jnp.full((EPAD - E - N,), N, jnp.int32)
    src2d = jnp.concatenate([edge_index[0], loops, pad]).reshape(NR, SUB)
    dst2d = jnp.concatenate([edge_index[1], loops, pad]).reshape(NR, SUB)
    emb_pad = jnp.pad(emb, ((0, NP - N), (0, 0)))

    tc0_out = _tc0(emb_pad, W1, as1, ad1)
    x_tabs, ast1, adt1 = tc0_out[0:8], tc0_out[8], tc0_out[9]
    e1, dpart1 = _sc_a(src2d, dst2d, ast1, adt1)
    dinv1 = _tc_recip(dpart1)
    att1 = _sc_a2(dst2d, e1, dinv1)
    outb1 = _sc_b_l1(src2d, dst2d, att1, *x_tabs)

    tc2_out = _tc2([outb1[u] for u in range(8)], b1, W2, as2, ad2)
    y_tabs, ast2, adt2 = tc2_out[0:4], tc2_out[4], tc2_out[5]
    e2, dpart2 = _sc_a(src2d, dst2d, ast2, adt2)
    dinv2 = _tc_recip(dpart2)
    att2 = _sc_a2(dst2d, e2, dinv2)
    outb2 = _sc_b_l2(src2d, dst2d, att2, *y_tabs)

    u2d = user_indices.reshape(NW, SUB)
    i2d = item_indices.reshape(NW, SUB)
    g = _sc_g(outb2[0], outb2[1], outb2[2], outb2[3], u2d, i2d)
    return _mlp_head(g[0:4], g[4:8], b2, fw1, fb1, fg, fbe, fw2, fb2,
                     pw1, pb1, pg, pbe, pw2, pb2, pw3, pb3)


# speedup vs baseline: 25.7547x; 1.1209x over previous
import jax
import jax.numpy as jnp
from jax import lax
from jax.experimental import pallas as pl
from jax.experimental.pallas import tpu as pltpu
from jax.experimental.pallas import tpu_sc as plsc

N = 50000
NP = 50048
EMB = 64
HID = 128
NC, NS = 2, 16
NW = NC * NS
SUB = 128
KS = 4
BE = SUB * KS
NBLK = 52
TPT = NBLK * KS
NR = NW * TPT
EPAD = NR * SUB
NPT16 = NP // NS

_mesh = plsc.VectorSubcoreMesh(core_axis_name="c", subcore_axis_name="s")
_sc_params = pltpu.CompilerParams(use_tc_tiling_on_sc=False,
                                  needs_layout_passes=False)


def _wid():
    return lax.axis_index("c") * NS + lax.axis_index("s")


def _sc_a_body(src2d, dst2d, ast, adt, e_out, dpart, *rest):
    idxs = rest[0:KS]
    idxd = rest[KS:2 * KS]
    srows, drows, ebuf, zbuf, sem, accum = rest[2 * KS:]
    c = lax.axis_index("c")
    s = lax.axis_index("s")
    wid = c * NS + s
    zero16 = jnp.zeros((16,), jnp.float32)

    @pl.loop(0, NPT16)
    def _(i):
        zbuf[i, :] = zero16

    pltpu.sync_copy(zbuf, accum.at[pl.ds(s * NPT16, NPT16)])
    plsc.subcore_barrier()

    r0 = wid * TPT

    @pl.loop(0, NBLK)
    def _(b):
        r = r0 + b * KS
        ebase = r * SUB
        for j in range(KS):
            pltpu.sync_copy(src2d.at[r + j], idxs[j])
            pltpu.sync_copy(dst2d.at[r + j], idxd[j])
            pltpu.async_copy(ast.at[idxs[j]],
                             srows.at[pl.ds(j * SUB, SUB)], sem).wait()
            pltpu.async_copy(adt.at[idxd[j]],
                             drows.at[pl.ds(j * SUB, SUB)], sem).wait()

        @pl.loop(0, BE)
        def _(i):
            xv = srows[i, :] + drows[i, :]
            xv = jnp.maximum(xv, xv * 0.2)
            ebuf[i, :] = jnp.exp(xv)

        pltpu.sync_copy(ebuf, e_out.at[pl.ds(ebase, BE)])
        for j in range(KS):
            pltpu.sync_copy(ebuf.at[pl.ds(j * SUB, SUB)],
                            accum.at[idxd[j]], add=True)

    plsc.subcore_barrier()
    for cc in range(NC):
        @pl.when(c == cc)
        def _():
            pltpu.sync_copy(accum.at[pl.ds(s * NPT16, NPT16)],
                            dpart.at[cc, pl.ds(s * NPT16, NPT16)])


_sc_a = pl.kernel(
    _sc_a_body,
    out_type=(jax.ShapeDtypeStruct((EPAD, 16), jnp.float32),
              jax.ShapeDtypeStruct((NC, NP, 16), jnp.float32)),
    mesh=_mesh,
    compiler_params=_sc_params,
    scratch_types=[pltpu.VMEM((SUB,), jnp.int32)] * (2 * KS) + [
        pltpu.VMEM((BE, 16), jnp.float32),
        pltpu.VMEM((BE, 16), jnp.float32),
        pltpu.VMEM((BE, 16), jnp.float32),
        pltpu.VMEM((NPT16, 16), jnp.float32),
        pltpu.SemaphoreType.DMA,
        pltpu.VMEM_SHARED((NP, 16), jnp.float32),
    ],
)


def _sc_a2_body(dst2d, e_in, dinvt, att_t, *rest):
    dsti = (rest[0:KS], rest[KS:2 * KS])
    o = 2 * KS
    e2d = rest[o:o + 2]
    drows = rest[o + 2:o + 4]
    attflat = rest[o + 4:o + 6]
    attcol = rest[o + 6:o + 8]
    si = rest[o + 8:o + 10]
    sg = rest[o + 10:o + 12]
    sa = rest[o + 12:o + 14]
    wid = _wid()
    iota = lax.iota(jnp.int32, 16)
    r0 = wid * TPT
    nb2 = NBLK // 2

    def idx_copies(b, p):
        r = r0 + b * KS
        for j in range(KS):
            yield dst2d.at[r + j], dsti[p][j], si[p]

    def start_idx(b, p):
        for a, d, m in idx_copies(b, p):
            pltpu.async_copy(a, d, m)

    def wait_idx(b, p):
        for a, d, m in idx_copies(b, p):
            pltpu.make_async_copy(a, d, m).wait()

    def gather_copies(b, p):
        ebase = (r0 + b * KS) * SUB
        yield e_in.at[pl.ds(ebase, BE)], e2d[p], sg[p]
        for j in range(KS):
            yield (dinvt.at[dsti[p][j]],
                   drows[p].at[pl.ds(j * SUB, SUB)], sg[p])

    def start_gathers(b, p):
        for a, d, m in gather_copies(b, p):
            pltpu.async_copy(a, d, m)

    def wait_gathers(b, p):
        for a, d, m in gather_copies(b, p):
            pltpu.make_async_copy(a, d, m).wait()

    def att_copies(b, p):
        ebase = (r0 + b * KS) * SUB
        for h in range(4):
            yield attcol[p].at[h], att_t.at[h, pl.ds(ebase, BE)], sa[p]

    def start_att(b, p):
        for a, d, m in att_copies(b, p):
            pltpu.async_copy(a, d, m)

    def wait_att(b, p):
        for a, d, m in att_copies(b, p):
            pltpu.make_async_copy(a, d, m).wait()

    def compute(p):
        @pl.loop(0, BE)
        def _(i):
            attflat[p][pl.ds(i * 16, 16)] = e2d[p][i, :] * drows[p][i, :]

        for h in range(4):
            @pl.loop(0, BE // 16)
            def _(g):
                idx = iota * 16 + (g * 256 + h)
                attcol[p][h, pl.ds(g * 16, 16)] = plsc.load_gather(
                    attflat[p], [idx])

    start_idx(0, 0)
    start_idx(1, 1)
    wait_idx(0, 0)
    start_gathers(0, 0)

    @pl.loop(0, nb2)
    def _(t):
        for p in (0, 1):
            b = 2 * t + p

            @pl.when(b + 1 < NBLK)
            def _():
                wait_idx(b + 1, 1 - p)
                start_gathers(b + 1, 1 - p)

            wait_gathers(b, p)

            @pl.when(t > 0)
            def _():
                wait_att(b - 2, p)

            @pl.when(b + 2 < NBLK)
            def _():
                start_idx(b + 2, p)

            compute(p)
            start_att(b, p)

    wait_att(NBLK - 2, 0)
    wait_att(NBLK - 1, 1)


_sc_a2 = pl.kernel(
    _sc_a2_body,
    out_type=jax.ShapeDtypeStruct((4, EPAD), jnp.float32),
    mesh=_mesh,
    compiler_params=_sc_params,
    scratch_types=(
        [pltpu.VMEM((SUB,), jnp.int32)] * (2 * KS)
        + [pltpu.VMEM((BE, 16), jnp.float32)] * 4
        + [pltpu.VMEM((BE * 16,), jnp.float32)] * 2
        + [pltpu.VMEM((4, BE), jnp.float32)] * 2
        + [pltpu.SemaphoreType.DMA] * 6
    ),
)


def _build_sc_b(upc, att_rows):
    n_units = NC * upc
    nrows_t = NPT16
    zr = nrows_t // 8
    tpt_b = NR // NS
    nblk_b = tpt_b // KS
    nb4 = nblk_b // 4

    def body(*args):
        src2d, dst2d, att_t = args[0:3]
        tables = args[3:3 + n_units]
        out_hbm = args[3 + n_units]
        rest = args[4 + n_units:]
        srci = rest[0:2]
        dsti = tuple(rest[2 + 4 * q:2 + 4 * q + KS] for q in range(4))
        o = 2 + 4 * KS
        attb = rest[o:o + 2]
        xrows = rest[o + 2:o + 4]
        msg = rest[o + 4:o + 6]
        zbuf = rest[o + 6]
        si = rest[o + 7:o + 9]
        sg = rest[o + 9:o + 11]
        ss = rest[o + 11:o + 13]
        accum = rest[o + 13]
        c = lax.axis_index("c")
        s = lax.axis_index("s")
        zero16 = jnp.zeros((16,), jnp.float32)

        @pl.loop(0, zr)
        def _(i):
            zbuf[i, :] = zero16

        r0 = s * tpt_b

        def idx_copies(b, p, dq):
            r = r0 + b * KS
            yield src2d.at[pl.ds(r, KS)], srci[p], si[p]
            for j in range(KS):
                yield dst2d.at[r + j], dsti[dq][j], si[p]

        def start_idx(b, p, dq):
            for a, d, m in idx_copies(b, p, dq):
                pltpu.async_copy(a, d, m)

        def wait_idx(b, p, dq):
            for a, d, m in idx_copies(b, p, dq):
                pltpu.make_async_copy(a, d, m).wait()

        def gather_copies(b, p, table, arow):
            r = r0 + b * KS
            yield att_t.at[arow, pl.ds(r * SUB, BE)], attb[p], sg[p]
            for j in range(KS):
                yield (table.at[srci[p].at[j]],
                       xrows[p].at[pl.ds(j * SUB, SUB)], sg[p])

        def start_gathers(b, p, table, arow):
            for a, d, m in gather_copies(b, p, table, arow):
                pltpu.async_copy(a, d, m)

        def wait_gathers(b, p, table, arow):
            for a, d, m in gather_copies(b, p, table, arow):
                pltpu.make_async_copy(a, d, m).wait()

        def start_scat(p, dq):
            for j in range(KS):
                pltpu.async_copy(msg[p].at[pl.ds(j * SUB, SUB)],
                                 accum.at[dsti[dq][j]], ss[p], add=True)

        def wait_scat(p, dq):
            for j in range(KS):
                pltpu.make_async_copy(msg[p].at[pl.ds(j * SUB, SUB)],
                                      accum.at[dsti[dq][j]], ss[p]).wait()

        def compute(p):
            @pl.loop(0, BE // 4)
            def _(i):
                for k4 in range(4):
                    e = i * 4 + k4
                    ab = plsc.load_gather(attb[p],
                                          [jnp.broadcast_to(e, (16,))])
                    msg[p][e, :] = xrows[p][e, :] * ab

        for cc in range(NC):
            @pl.when(c == cc)
            def _():
                for jj in range(upc):
                    u = cc * upc + jj
                    table = tables[u]
                    arow = att_rows[u]
                    for q in range(8):
                        pltpu.sync_copy(
                            zbuf, accum.at[pl.ds(s * nrows_t + q * zr, zr)])
                    plsc.subcore_barrier()

                    start_idx(0, 0, 0)
                    start_idx(1, 1, 1)
                    wait_idx(0, 0, 0)
                    start_gathers(0, 0, table, arow)

                    @pl.loop(0, nb4)
                    def _(t2):
                        for q in range(4):
                            b = 4 * t2 + q
                            p = q % 2

                            @pl.when(b + 1 < nblk_b)
                            def _():
                                wait_idx(b + 1, 1 - p, (q + 1) % 4)
                                start_gathers(b + 1, 1 - p, table, arow)

                            wait_gathers(b, p, table, arow)

                            if q >= 2:
                                wait_scat(p, q - 2)
                            else:
                                @pl.when(t2 > 0)
                                def _():
                                    wait_scat(p, q + 2)

                            @pl.when(b + 2 < nblk_b)
                            def _():
                                start_idx(b + 2, p, (q + 2) % 4)

                            compute(p)
                            start_scat(p, q)

                    wait_scat(0, 2)
                    wait_scat(1, 3)
                    plsc.subcore_barrier()
                    pltpu.sync_copy(accum.at[pl.ds(s * nrows_t, nrows_t)],
                                    out_hbm.at[u, pl.ds(s * nrows_t, nrows_t)])
                    plsc.subcore_barrier()

    return pl.kernel(
        body,
        out_type=jax.ShapeDtypeStruct((n_units, NP, 16), jnp.float32),
        mesh=_mesh,
        compiler_params=_sc_params,
        scratch_types=(
            [pltpu.VMEM((KS, SUB), jnp.int32)] * 2
            + [pltpu.VMEM((SUB,), jnp.int32)] * (4 * KS)
            + [pltpu.VMEM((BE,), jnp.float32)] * 2
            + [pltpu.VMEM((BE, 16), jnp.float32)] * 4
            + [pltpu.VMEM((zr, 16), jnp.float32)]
            + [pltpu.SemaphoreType.DMA] * 6
            + [pltpu.VMEM_SHARED((NP, 16), jnp.float32)]
        ),
    )


_sc_b_l1 = _build_sc_b(4, (0, 0, 1, 1, 2, 2, 3, 3))
_sc_b_l2 = _build_sc_b(2, (0, 0, 0, 0))


def _sc_g_body(*args):
    tabs = args[0:4]
    u2d, i2d = args[4:6]
    outs = args[6:14]
    idxb, rows, sem = args[14:]
    wid = _wid()
    for t in range(4):
        pltpu.sync_copy(u2d.at[wid], idxb)
        pltpu.async_copy(tabs[t].at[idxb], rows, sem).wait()
        pltpu.sync_copy(rows, outs[t].at[pl.ds(wid * SUB, SUB)])
        pltpu.sync_copy(i2d.at[wid], idxb)
        pltpu.async_copy(tabs[t].at[idxb], rows, sem).wait()
        pltpu.sync_copy(rows, outs[4 + t].at[pl.ds(wid * SUB, SUB)])


_sc_g = pl.kernel(
    _sc_g_body,
    out_type=tuple(jax.ShapeDtypeStruct((4096, 16), jnp.float32)
                   for _ in range(8)),
    mesh=_mesh,
    compiler_params=_sc_params,
    scratch_types=[
        pltpu.VMEM((SUB,), jnp.int32),
        pltpu.VMEM((SUB, 16), jnp.float32),
        pltpu.SemaphoreType.DMA,
    ],
)


def _tc0_body(emb_ref, w1_ref, as1_ref, ad1_ref, *out_refs):
    xh = jnp.dot(emb_ref[...], w1_ref[...], preferred_element_type=jnp.float32)
    as1 = as1_ref[...]
    ad1 = ad1_ref[...]
    for u in range(8):
        out_refs[u][...] = xh[:, 16 * u:16 * u + 16]
    a_s, a_d = [], []
    for h in range(4):
        seg = xh[:, 32 * h:32 * h + 32]
        a_s.append(jnp.sum(seg * as1[h][None, :], axis=1, keepdims=True))
        a_d.append(jnp.sum(seg * ad1[h][None, :], axis=1, keepdims=True))
    z = jnp.zeros((xh.shape[0], 12), jnp.float32)
    out_refs[8][...] = jnp.concatenate(a_s + [z], axis=1)
    out_refs[9][...] = jnp.concatenate(a_d + [z], axis=1)


def _tc0(emb_pad, W1, as1, ad1):
    BN = 128
    full = lambda a: pl.BlockSpec(a.shape, lambda i: tuple(0 for _ in a.shape))
    bs16 = pl.BlockSpec((BN, 16), lambda i: (i, 0))
    return pl.pallas_call(
        _tc0_body,
        grid=(NP // BN,),
        in_specs=[pl.BlockSpec((BN, EMB), lambda i: (i, 0)),
                  full(W1), full(as1), full(ad1)],
        out_specs=[bs16] * 10,
        out_shape=[jax.ShapeDtypeStruct((NP, 16), jnp.float32)] * 10,
    )(emb_pad, W1, as1, ad1)


def _tc_recip_body(dp_ref, out_ref):
    out_ref[...] = 1.0 / (dp_ref[0] + dp_ref[1] + 1e-16)


def _tc_recip(dpart):
    dpv = dpart.reshape(NC, NP * 16 // 128, 128)
    R = dpv.shape[1]
    BN = 16
    out = pl.pallas_call(
        _tc_recip_body,
        grid=(R // BN,),
        in_specs=[pl.BlockSpec((NC, BN, 128), lambda i: (0, i, 0))],
        out_specs=pl.BlockSpec((BN, 128), lambda i: (i, 0)),
        out_shape=jax.ShapeDtypeStruct((R, 128), jnp.float32),
    )(dpv)
    return out.reshape(NP, 16)


def _tc2_body(*refs):
    in_refs = refs[0:8]
    b1_ref, w2_ref, as2_ref, ad2_ref = refs[8:12]
    out_refs = refs[12:]
    o = (jnp.concatenate([r[...] for r in in_refs], axis=1)
         + b1_ref[...][None, :])
    h1 = jnp.where(o > 0, o, jnp.exp(jnp.minimum(o, 0.0)) - 1.0)
    xh2 = jnp.dot(h1, w2_ref[...], preferred_element_type=jnp.float32)
    for u in range(4):
        out_refs[u][...] = xh2[:, 16 * u:16 * u + 16]
    z = jnp.zeros((xh2.shape[0], 15), jnp.float32)
    a_s = jnp.sum(xh2 * as2_ref[...][0][None, :], axis=1, keepdims=True)
    a_d = jnp.sum(xh2 * ad2_ref[...][0][None, :], axis=1, keepdims=True)
    out_refs[4][...] = jnp.concatenate([a_s, z], axis=1)
    out_refs[5][...] = jnp.concatenate([a_d, z], axis=1)


def _tc2(o_parts, b1, W2, as2, ad2):
    BN = 128
    full = lambda a: pl.BlockSpec(a.shape, lambda i: tuple(0 for _ in a.shape))
    bs16 = pl.BlockSpec((BN, 16), lambda i: (i, 0))
    return pl.pallas_call(
        _tc2_body,
        grid=(NP // BN,),
        in_specs=[bs16] * 8 + [full(b1), full(W2), full(as2), full(ad2)],
        out_specs=[bs16] * 6,
        out_shape=[jax.ShapeDtypeStruct((NP, 16), jnp.float32)] * 6,
    )(*o_parts, b1, W2, as2, ad2)


_BN_SCALE = 1.0 / (1.0 + 1e-5) ** 0.5


def _head_body(*refs):
    ue_refs = refs[0:4]
    ie_refs = refs[4:8]
    (b2_ref, fw1_ref, fb1_ref, fg_ref, fbe_ref, fw2_ref, fb2_ref, pw1_ref,
     pb1_ref, pg_ref, pbe_ref, pw2_ref, pb2_ref, pw3_ref, pb3_ref,
     out_ref) = refs[8:]
    b2 = b2_ref[...]
    ue = jnp.concatenate([r[...] for r in ue_refs], axis=1) + b2[None, :]
    ie = jnp.concatenate([r[...] for r in ie_refs], axis=1) + b2[None, :]
    fw1s = fw1_ref[...]
    fb1 = fb1_ref[...]
    fscale = fg_ref[...] * _BN_SCALE
    fbe = fbe_ref[...]
    fw2 = fw2_ref[...]
    fb2 = fb2_ref[...]

    def fusion(e):
        z = jnp.dot(e, fw1s, preferred_element_type=jnp.float32) + fb1
        z = jnp.maximum(z * fscale + fbe, 0.0)
        return jnp.dot(z, fw2, preferred_element_type=jnp.float32) + fb2

    uef = fusion(ue)
    ief = fusion(ie)
    pw1 = pw1_ref[...]
    z = (jnp.dot(uef, pw1[:64], preferred_element_type=jnp.float32)
         + jnp.dot(ief, pw1[64:], preferred_element_type=jnp.float32)
         + pb1_ref[...])
    z = jnp.maximum(z * (pg_ref[...] * _BN_SCALE) + pbe_ref[...], 0.0)
    z = jnp.maximum(jnp.dot(z, pw2_ref[...], preferred_element_type=jnp.float32)
                    + pb2_ref[...], 0.0)
    out_ref[...] = (jnp.dot(z, pw3_ref[...], preferred_element_type=jnp.float32)
                    + pb3_ref[...])


def _mlp_head(ue_parts, ie_parts, b2, fw1, fb1, fg, fbe, fw2, fb2,
              pw1, pb1, pg, pbe, pw2, pb2, pw3, pb3):
    B = ue_parts[0].shape[0]
    BT = 1024
    fw1s = fw1[:64] + fw1[64:]
    bspec = pl.BlockSpec((BT, 16), lambda i: (i, 0))
    full = lambda a: pl.BlockSpec(a.shape, lambda i: tuple(0 for _ in a.shape))
    out = pl.pallas_call(
        _head_body,
        grid=(B // BT,),
        in_specs=[bspec] * 8
                 + [full(a) for a in (b2, fw1s, fb1, fg, fbe, fw2, fb2, pw1,
                                      pb1, pg, pbe, pw2, pb2, pw3, pb3)],
        out_specs=pl.BlockSpec((BT, 1), lambda i: (i, 0)),
        out_shape=jax.ShapeDtypeStruct((B, 1), jnp.float32),
    )(*ue_parts, *ie_parts, b2, fw1s, fb1, fg, fbe, fw2, fb2,
      pw1, pb1, pg, pbe, pw2, pb2, pw3, pb3)
    return out[:, 0]


def kernel(x, edge_index, user_indices, item_indices, emb, W1, as1, ad1, b1,
           W2, as2, ad2, b2, fw1, fb1, fg, fbe, fw2, fb2,
           pw1, pb1, pg, pbe, pw2, pb2, pw3, pb3):
    E = edge_index.shape[1]
    loops = jnp.arange(N, dtype=jnp.int32)
    pad = jnp.full((EPAD - E - N,), N, jnp.int32)
    src2d = jnp.concatenate([edge_index[0], loops, pad]).reshape(NR, SUB)
    dst2d = jnp.concatenate([edge_index[1], loops, pad]).reshape(NR, SUB)
    emb_pad = jnp.pad(emb, ((0, NP - N), (0, 0)))

    tc0_out = _tc0(emb_pad, W1, as1, ad1)
    x_tabs, ast1, adt1 = tc0_out[0:8], tc0_out[8], tc0_out[9]
    e1, dpart1 = _sc_a(src2d, dst2d, ast1, adt1)
    dinv1 = _tc_recip(dpart1)
    att1 = _sc_a2(dst2d, e1, dinv1)
    outb1 = _sc_b_l1(src2d, dst2d, att1, *x_tabs)

    tc2_out = _tc2([outb1[u] for u in range(8)], b1, W2, as2, ad2)
    y_tabs, ast2, adt2 = tc2_out[0:4], tc2_out[4], tc2_out[5]
    e2, dpart2 = _sc_a(src2d, dst2d, ast2, adt2)
    dinv2 = _tc_recip(dpart2)
    att2 = _sc_a2(dst2d, e2, dinv2)
    outb2 = _sc_b_l2(src2d, dst2d, att2, *y_tabs)

    u2d = user_indices.reshape(NW, SUB)
    i2d = item_indices.reshape(NW, SUB)
    g = _sc_g(outb2[0], outb2[1], outb2[2], outb2[3], u2d, i2d)
    return _mlp_head(g[0:4], g[4:8], b2, fw1, fb1, fg, fbe, fw2, fb2,
                     pw1, pb1, pg, pbe, pw2, pb2, pw3, pb3)


# speedup vs baseline: 32.2489x; 1.2522x over previous
import jax
import jax.numpy as jnp
from jax import lax
from jax.experimental import pallas as pl
from jax.experimental.pallas import tpu as pltpu
from jax.experimental.pallas import tpu_sc as plsc

N = 50000
NP = 50048
EMB = 64
HID = 128
NC, NS = 2, 16
NW = NC * NS
SUB = 128
KS = 4
BE = SUB * KS
NBLK = 52
TPT = NBLK * KS
NR = NW * TPT
EPAD = NR * SUB
NPT16 = NP // NS

_mesh = plsc.VectorSubcoreMesh(core_axis_name="c", subcore_axis_name="s")
_sc_params = pltpu.CompilerParams(use_tc_tiling_on_sc=False,
                                  needs_layout_passes=False)


def _wid():
    return lax.axis_index("c") * NS + lax.axis_index("s")


def _sc_a_body(src2d, dst2d, ast, adt, e_out, dpart, *rest):
    srci = rest[0:2]
    dsti = tuple(rest[2 + KS * q:2 + KS * q + KS] for q in range(4))
    o = 2 + 4 * KS
    srows = rest[o:o + 2]
    drows = rest[o + 2:o + 4]
    ebuf = rest[o + 4:o + 6]
    zbuf = rest[o + 6]
    si = rest[o + 7:o + 9]
    sg = rest[o + 9:o + 11]
    ss = rest[o + 11:o + 13]
    accum = rest[o + 13]
    c = lax.axis_index("c")
    s = lax.axis_index("s")
    wid = c * NS + s
    zero16 = jnp.zeros((16,), jnp.float32)
    zr = NPT16 // 8

    @pl.loop(0, zr)
    def _(i):
        zbuf[i, :] = zero16

    for q in range(8):
        pltpu.sync_copy(zbuf, accum.at[pl.ds(s * NPT16 + q * zr, zr)])
    plsc.subcore_barrier()

    r0 = wid * TPT
    nb4 = NBLK // 4

    def idx_copies(b, p, dq):
        r = r0 + b * KS
        yield src2d.at[pl.ds(r, KS)], srci[p], si[p]
        for j in range(KS):
            yield dst2d.at[r + j], dsti[dq][j], si[p]

    def start_idx(b, p, dq):
        for a, d, m in idx_copies(b, p, dq):
            pltpu.async_copy(a, d, m)

    def wait_idx(b, p, dq):
        for a, d, m in idx_copies(b, p, dq):
            pltpu.make_async_copy(a, d, m).wait()

    def gather_copies(p, dq):
        for j in range(KS):
            yield (ast.at[srci[p].at[j]],
                   srows[p].at[pl.ds(j * SUB, SUB)], sg[p])
            yield (adt.at[dsti[dq][j]],
                   drows[p].at[pl.ds(j * SUB, SUB)], sg[p])

    def start_gathers(p, dq):
        for a, d, m in gather_copies(p, dq):
            pltpu.async_copy(a, d, m)

    def wait_gathers(p, dq):
        for a, d, m in gather_copies(p, dq):
            pltpu.make_async_copy(a, d, m).wait()

    def scat_copies(p, dq):
        for j in range(KS):
            yield (ebuf[p].at[pl.ds(j * SUB, SUB)], accum.at[dsti[dq][j]],
                   ss[p])

    def start_scat(p, dq):
        for a, d, m in scat_copies(p, dq):
            pltpu.async_copy(a, d, m, add=True)

    def wait_scat(p, dq):
        for a, d, m in scat_copies(p, dq):
            pltpu.make_async_copy(a, d, m).wait()

    def compute(p):
        @pl.loop(0, BE)
        def _(i):
            xv = srows[p][i, :] + drows[p][i, :]
            xv = jnp.maximum(xv, xv * 0.2)
            ebuf[p][i, :] = jnp.exp(xv)

    start_idx(0, 0, 0)
    start_idx(1, 1, 1)
    wait_idx(0, 0, 0)
    start_gathers(0, 0)

    @pl.loop(0, nb4)
    def _(t2):
        for q in range(4):
            b = 4 * t2 + q
            p = q % 2

            @pl.when(b + 1 < NBLK)
            def _():
                wait_idx(b + 1, 1 - p, (q + 1) % 4)
                start_gathers(1 - p, (q + 1) % 4)

            wait_gathers(p, q)

            if q >= 2:
                wait_scat(p, q - 2)
            else:
                @pl.when(t2 > 0)
                def _():
                    wait_scat(p, q + 2)

            @pl.when(b + 2 < NBLK)
            def _():
                start_idx(b + 2, p, (q + 2) % 4)

            compute(p)
            pltpu.sync_copy(ebuf[p], e_out.at[pl.ds((r0 + b * KS) * SUB, BE)])
            start_scat(p, q)

    wait_scat(0, 2)
    wait_scat(1, 3)
    plsc.subcore_barrier()
    for cc in range(NC):
        @pl.when(c == cc)
        def _():
            pltpu.sync_copy(accum.at[pl.ds(s * NPT16, NPT16)],
                            dpart.at[cc, pl.ds(s * NPT16, NPT16)])


_sc_a = pl.kernel(
    _sc_a_body,
    out_type=(jax.ShapeDtypeStruct((EPAD, 16), jnp.float32),
              jax.ShapeDtypeStruct((NC, NP, 16), jnp.float32)),
    mesh=_mesh,
    compiler_params=_sc_params,
    scratch_types=(
        [pltpu.VMEM((KS, SUB), jnp.int32)] * 2
        + [pltpu.VMEM((SUB,), jnp.int32)] * (4 * KS)
        + [pltpu.VMEM((BE, 16), jnp.float32)] * 6
        + [pltpu.VMEM((NPT16 // 8, 16), jnp.float32)]
        + [pltpu.SemaphoreType.DMA] * 6
        + [pltpu.VMEM_SHARED((NP, 16), jnp.float32)]
    ),
)


def _sc_a2_body(dst2d, e_in, dinvt, att_t, *rest):
    dsti = (rest[0:KS], rest[KS:2 * KS])
    o = 2 * KS
    e2d = rest[o:o + 2]
    drows = rest[o + 2:o + 4]
    attflat = rest[o + 4:o + 6]
    attcol = rest[o + 6:o + 8]
    si = rest[o + 8:o + 10]
    sg = rest[o + 10:o + 12]
    sa = rest[o + 12:o + 14]
    wid = _wid()
    iota = lax.iota(jnp.int32, 16)
    r0 = wid * TPT
    nb2 = NBLK // 2

    def idx_copies(b, p):
        r = r0 + b * KS
        for j in range(KS):
            yield dst2d.at[r + j], dsti[p][j], si[p]

    def start_idx(b, p):
        for a, d, m in idx_copies(b, p):
            pltpu.async_copy(a, d, m)

    def wait_idx(b, p):
        for a, d, m in idx_copies(b, p):
            pltpu.make_async_copy(a, d, m).wait()

    def gather_copies(b, p):
        ebase = (r0 + b * KS) * SUB
        yield e_in.at[pl.ds(ebase, BE)], e2d[p], sg[p]
        for j in range(KS):
            yield (dinvt.at[dsti[p][j]],
                   drows[p].at[pl.ds(j * SUB, SUB)], sg[p])

    def start_gathers(b, p):
        for a, d, m in gather_copies(b, p):
            pltpu.async_copy(a, d, m)

    def wait_gathers(b, p):
        for a, d, m in gather_copies(b, p):
            pltpu.make_async_copy(a, d, m).wait()

    def att_copies(b, p):
        ebase = (r0 + b * KS) * SUB
        for h in range(4):
            yield attcol[p].at[h], att_t.at[h, pl.ds(ebase, BE)], sa[p]

    def start_att(b, p):
        for a, d, m in att_copies(b, p):
            pltpu.async_copy(a, d, m)

    def wait_att(b, p):
        for a, d, m in att_copies(b, p):
            pltpu.make_async_copy(a, d, m).wait()

    def compute(p):
        @pl.loop(0, BE)
        def _(i):
            attflat[p][pl.ds(i * 16, 16)] = e2d[p][i, :] * drows[p][i, :]

        for h in range(4):
            @pl.loop(0, BE // 16)
            def _(g):
                idx = iota * 16 + (g * 256 + h)
                attcol[p][h, pl.ds(g * 16, 16)] = plsc.load_gather(
                    attflat[p], [idx])

    start_idx(0, 0)
    start_idx(1, 1)
    wait_idx(0, 0)
    start_gathers(0, 0)

    @pl.loop(0, nb2)
    def _(t):
        for p in (0, 1):
            b = 2 * t + p

            @pl.when(b + 1 < NBLK)
            def _():
                wait_idx(b + 1, 1 - p)
                start_gathers(b + 1, 1 - p)

            wait_gathers(b, p)

            @pl.when(t > 0)
            def _():
                wait_att(b - 2, p)

            @pl.when(b + 2 < NBLK)
            def _():
                start_idx(b + 2, p)

            compute(p)
            start_att(b, p)

    wait_att(NBLK - 2, 0)
    wait_att(NBLK - 1, 1)


_sc_a2 = pl.kernel(
    _sc_a2_body,
    out_type=jax.ShapeDtypeStruct((4, EPAD), jnp.float32),
    mesh=_mesh,
    compiler_params=_sc_params,
    scratch_types=(
        [pltpu.VMEM((SUB,), jnp.int32)] * (2 * KS)
        + [pltpu.VMEM((BE, 16), jnp.float32)] * 4
        + [pltpu.VMEM((BE * 16,), jnp.float32)] * 2
        + [pltpu.VMEM((4, BE), jnp.float32)] * 2
        + [pltpu.SemaphoreType.DMA] * 6
    ),
)


def _build_sc_b(upc, att_rows):
    n_units = NC * upc
    nrows_t = NPT16
    zr = nrows_t // 8
    tpt_b = NR // NS
    nblk_b = tpt_b // KS
    nb4 = nblk_b // 4

    def body(*args):
        src2d, dst2d, att_t = args[0:3]
        tables = args[3:3 + n_units]
        out_hbm = args[3 + n_units]
        rest = args[4 + n_units:]
        srci = rest[0:2]
        dsti = tuple(rest[2 + 4 * q:2 + 4 * q + KS] for q in range(4))
        o = 2 + 4 * KS
        attb = rest[o:o + 2]
        xrows = rest[o + 2:o + 4]
        msg = rest[o + 4:o + 6]
        zbuf = rest[o + 6]
        si = rest[o + 7:o + 9]
        sg = rest[o + 9:o + 11]
        ss = rest[o + 11:o + 13]
        accum = rest[o + 13]
        c = lax.axis_index("c")
        s = lax.axis_index("s")
        zero16 = jnp.zeros((16,), jnp.float32)

        @pl.loop(0, zr)
        def _(i):
            zbuf[i, :] = zero16

        r0 = s * tpt_b

        def idx_copies(b, p, dq):
            r = r0 + b * KS
            yield src2d.at[pl.ds(r, KS)], srci[p], si[p]
            for j in range(KS):
                yield dst2d.at[r + j], dsti[dq][j], si[p]

        def start_idx(b, p, dq):
            for a, d, m in idx_copies(b, p, dq):
                pltpu.async_copy(a, d, m)

        def wait_idx(b, p, dq):
            for a, d, m in idx_copies(b, p, dq):
                pltpu.make_async_copy(a, d, m).wait()

        def gather_copies(b, p, table, arow):
            r = r0 + b * KS
            yield att_t.at[arow, pl.ds(r * SUB, BE)], attb[p], sg[p]
            for j in range(KS):
                yield (table.at[srci[p].at[j]],
                       xrows[p].at[pl.ds(j * SUB, SUB)], sg[p])

        def start_gathers(b, p, table, arow):
            for a, d, m in gather_copies(b, p, table, arow):
                pltpu.async_copy(a, d, m)

        def wait_gathers(b, p, table, arow):
            for a, d, m in gather_copies(b, p, table, arow):
                pltpu.make_async_copy(a, d, m).wait()

        def start_scat(p, dq):
            for j in range(KS):
                pltpu.async_copy(msg[p].at[pl.ds(j * SUB, SUB)],
                                 accum.at[dsti[dq][j]], ss[p], add=True)

        def wait_scat(p, dq):
            for j in range(KS):
                pltpu.make_async_copy(msg[p].at[pl.ds(j * SUB, SUB)],
                                      accum.at[dsti[dq][j]], ss[p]).wait()

        def compute(p):
            @pl.loop(0, BE // 4)
            def _(i):
                for k4 in range(4):
                    e = i * 4 + k4
                    ab = plsc.load_gather(attb[p],
                                          [jnp.broadcast_to(e, (16,))])
                    msg[p][e, :] = xrows[p][e, :] * ab

        for cc in range(NC):
            @pl.when(c == cc)
            def _():
                for jj in range(upc):
                    u = cc * upc + jj
                    table = tables[u]
                    arow = att_rows[u]
                    for q in range(8):
                        pltpu.sync_copy(
                            zbuf, accum.at[pl.ds(s * nrows_t + q * zr, zr)])
                    plsc.subcore_barrier()

                    start_idx(0, 0, 0)
                    start_idx(1, 1, 1)
                    wait_idx(0, 0, 0)
                    start_gathers(0, 0, table, arow)

                    @pl.loop(0, nb4)
                    def _(t2):
                        for q in range(4):
                            b = 4 * t2 + q
                            p = q % 2

                            @pl.when(b + 1 < nblk_b)
                            def _():
                                wait_idx(b + 1, 1 - p, (q + 1) % 4)
                                start_gathers(b + 1, 1 - p, table, arow)

                            wait_gathers(b, p, table, arow)

                            if q >= 2:
                                wait_scat(p, q - 2)
                            else:
                                @pl.when(t2 > 0)
                                def _():
                                    wait_scat(p, q + 2)

                            @pl.when(b + 2 < nblk_b)
                            def _():
                                start_idx(b + 2, p, (q + 2) % 4)

                            compute(p)
                            start_scat(p, q)

                    wait_scat(0, 2)
                    wait_scat(1, 3)
                    plsc.subcore_barrier()
                    pltpu.sync_copy(accum.at[pl.ds(s * nrows_t, nrows_t)],
                                    out_hbm.at[u, pl.ds(s * nrows_t, nrows_t)])
                    plsc.subcore_barrier()

    return pl.kernel(
        body,
        out_type=jax.ShapeDtypeStruct((n_units, NP, 16), jnp.float32),
        mesh=_mesh,
        compiler_params=_sc_params,
        scratch_types=(
            [pltpu.VMEM((KS, SUB), jnp.int32)] * 2
            + [pltpu.VMEM((SUB,), jnp.int32)] * (4 * KS)
            + [pltpu.VMEM((BE,), jnp.float32)] * 2
            + [pltpu.VMEM((BE, 16), jnp.float32)] * 4
            + [pltpu.VMEM((zr, 16), jnp.float32)]
            + [pltpu.SemaphoreType.DMA] * 6
            + [pltpu.VMEM_SHARED((NP, 16), jnp.float32)]
        ),
    )


_sc_b_l1 = _build_sc_b(4, (0, 0, 1, 1, 2, 2, 3, 3))
_sc_b_l2 = _build_sc_b(2, (0, 0, 0, 0))


def _sc_g_body(*args):
    tabs = args[0:4]
    u2d, i2d = args[4:6]
    outs = args[6:14]
    idxb, rows, sem = args[14:]
    wid = _wid()
    for t in range(4):
        pltpu.sync_copy(u2d.at[wid], idxb)
        pltpu.async_copy(tabs[t].at[idxb], rows, sem).wait()
        pltpu.sync_copy(rows, outs[t].at[pl.ds(wid * SUB, SUB)])
        pltpu.sync_copy(i2d.at[wid], idxb)
        pltpu.async_copy(tabs[t].at[idxb], rows, sem).wait()
        pltpu.sync_copy(rows, outs[4 + t].at[pl.ds(wid * SUB, SUB)])


_sc_g = pl.kernel(
    _sc_g_body,
    out_type=tuple(jax.ShapeDtypeStruct((4096, 16), jnp.float32)
                   for _ in range(8)),
    mesh=_mesh,
    compiler_params=_sc_params,
    scratch_types=[
        pltpu.VMEM((SUB,), jnp.int32),
        pltpu.VMEM((SUB, 16), jnp.float32),
        pltpu.SemaphoreType.DMA,
    ],
)


def _tc0_body(emb_ref, w1_ref, as1_ref, ad1_ref, *out_refs):
    xh = jnp.dot(emb_ref[...], w1_ref[...], preferred_element_type=jnp.float32)
    as1 = as1_ref[...]
    ad1 = ad1_ref[...]
    for u in range(8):
        out_refs[u][...] = xh[:, 16 * u:16 * u + 16]
    a_s, a_d = [], []
    for h in range(4):
        seg = xh[:, 32 * h:32 * h + 32]
        a_s.append(jnp.sum(seg * as1[h][None, :], axis=1, keepdims=True))
        a_d.append(jnp.sum(seg * ad1[h][None, :], axis=1, keepdims=True))
    z = jnp.zeros((xh.shape[0], 12), jnp.float32)
    out_refs[8][...] = jnp.concatenate(a_s + [z], axis=1)
    out_refs[9][...] = jnp.concatenate(a_d + [z], axis=1)


def _tc0(emb_pad, W1, as1, ad1):
    BN = 128
    full = lambda a: pl.BlockSpec(a.shape, lambda i: tuple(0 for _ in a.shape))
    bs16 = pl.BlockSpec((BN, 16), lambda i: (i, 0))
    return pl.pallas_call(
        _tc0_body,
        grid=(NP // BN,),
        in_specs=[pl.BlockSpec((BN, EMB), lambda i: (i, 0)),
                  full(W1), full(as1), full(ad1)],
        out_specs=[bs16] * 10,
        out_shape=[jax.ShapeDtypeStruct((NP, 16), jnp.float32)] * 10,
    )(emb_pad, W1, as1, ad1)


def _tc_recip_body(dp_ref, out_ref):
    out_ref[...] = 1.0 / (dp_ref[0] + dp_ref[1] + 1e-16)


def _tc_recip(dpart):
    dpv = dpart.reshape(NC, NP * 16 // 128, 128)
    R = dpv.shape[1]
    BN = 16
    out = pl.pallas_call(
        _tc_recip_body,
        grid=(R // BN,),
        in_specs=[pl.BlockSpec((NC, BN, 128), lambda i: (0, i, 0))],
        out_specs=pl.BlockSpec((BN, 128), lambda i: (i, 0)),
        out_shape=jax.ShapeDtypeStruct((R, 128), jnp.float32),
    )(dpv)
    return out.reshape(NP, 16)


def _tc2_body(*refs):
    in_refs = refs[0:8]
    b1_ref, w2_ref, as2_ref, ad2_ref = refs[8:12]
    out_refs = refs[12:]
    o = (jnp.concatenate([r[...] for r in in_refs], axis=1)
         + b1_ref[...][None, :])
    h1 = jnp.where(o > 0, o, jnp.exp(jnp.minimum(o, 0.0)) - 1.0)
    xh2 = jnp.dot(h1, w2_ref[...], preferred_element_type=jnp.float32)
    for u in range(4):
        out_refs[u][...] = xh2[:, 16 * u:16 * u + 16]
    z = jnp.zeros((xh2.shape[0], 15), jnp.float32)
    a_s = jnp.sum(xh2 * as2_ref[...][0][None, :], axis=1, keepdims=True)
    a_d = jnp.sum(xh2 * ad2_ref[...][0][None, :], axis=1, keepdims=True)
    out_refs[4][...] = jnp.concatenate([a_s, z], axis=1)
    out_refs[5][...] = jnp.concatenate([a_d, z], axis=1)


def _tc2(o_parts, b1, W2, as2, ad2):
    BN = 128
    full = lambda a: pl.BlockSpec(a.shape, lambda i: tuple(0 for _ in a.shape))
    bs16 = pl.BlockSpec((BN, 16), lambda i: (i, 0))
    return pl.pallas_call(
        _tc2_body,
        grid=(NP // BN,),
        in_specs=[bs16] * 8 + [full(b1), full(W2), full(as2), full(ad2)],
        out_specs=[bs16] * 6,
        out_shape=[jax.ShapeDtypeStruct((NP, 16), jnp.float32)] * 6,
    )(*o_parts, b1, W2, as2, ad2)


_BN_SCALE = 1.0 / (1.0 + 1e-5) ** 0.5


def _head_body(*refs):
    ue_refs = refs[0:4]
    ie_refs = refs[4:8]
    (b2_ref, fw1_ref, fb1_ref, fg_ref, fbe_ref, fw2_ref, fb2_ref, pw1_ref,
     pb1_ref, pg_ref, pbe_ref, pw2_ref, pb2_ref, pw3_ref, pb3_ref,
     out_ref) = refs[8:]
    b2 = b2_ref[...]
    ue = jnp.concatenate([r[...] for r in ue_refs], axis=1) + b2[None, :]
    ie = jnp.concatenate([r[...] for r in ie_refs], axis=1) + b2[None, :]
    fw1s = fw1_ref[...]
    fb1 = fb1_ref[...]
    fscale = fg_ref[...] * _BN_SCALE
    fbe = fbe_ref[...]
    fw2 = fw2_ref[...]
    fb2 = fb2_ref[...]

    def fusion(e):
        z = jnp.dot(e, fw1s, preferred_element_type=jnp.float32) + fb1
        z = jnp.maximum(z * fscale + fbe, 0.0)
        return jnp.dot(z, fw2, preferred_element_type=jnp.float32) + fb2

    uef = fusion(ue)
    ief = fusion(ie)
    pw1 = pw1_ref[...]
    z = (jnp.dot(uef, pw1[:64], preferred_element_type=jnp.float32)
         + jnp.dot(ief, pw1[64:], preferred_element_type=jnp.float32)
         + pb1_ref[...])
    z = jnp.maximum(z * (pg_ref[...] * _BN_SCALE) + pbe_ref[...], 0.0)
    z = jnp.maximum(jnp.dot(z, pw2_ref[...], preferred_element_type=jnp.float32)
                    + pb2_ref[...], 0.0)
    out_ref[...] = (jnp.dot(z, pw3_ref[...], preferred_element_type=jnp.float32)
                    + pb3_ref[...])


def _mlp_head(ue_parts, ie_parts, b2, fw1, fb1, fg, fbe, fw2, fb2,
              pw1, pb1, pg, pbe, pw2, pb2, pw3, pb3):
    B = ue_parts[0].shape[0]
    BT = 1024
    fw1s = fw1[:64] + fw1[64:]
    bspec = pl.BlockSpec((BT, 16), lambda i: (i, 0))
    full = lambda a: pl.BlockSpec(a.shape, lambda i: tuple(0 for _ in a.shape))
    out = pl.pallas_call(
        _head_body,
        grid=(B // BT,),
        in_specs=[bspec] * 8
                 + [full(a) for a in (b2, fw1s, fb1, fg, fbe, fw2, fb2, pw1,
                                      pb1, pg, pbe, pw2, pb2, pw3, pb3)],
        out_specs=pl.BlockSpec((BT, 1), lambda i: (i, 0)),
        out_shape=jax.ShapeDtypeStruct((B, 1), jnp.float32),
    )(*ue_parts, *ie_parts, b2, fw1s, fb1, fg, fbe, fw2, fb2,
      pw1, pb1, pg, pbe, pw2, pb2, pw3, pb3)
    return out[:, 0]


def kernel(x, edge_index, user_indices, item_indices, emb, W1, as1, ad1, b1,
           W2, as2, ad2, b2, fw1, fb1, fg, fbe, fw2, fb2,
           pw1, pb1, pg, pbe, pw2, pb2, pw3, pb3):
    E = edge_index.shape[1]
    loops = jnp.arange(N, dtype=jnp.int32)
    pad = jnp.full((EPAD - E - N,), N, jnp.int32)
    src2d = jnp.concatenate([edge_index[0], loops, pad]).reshape(NR, SUB)
    dst2d = jnp.concatenate([edge_index[1], loops, pad]).reshape(NR, SUB)
    emb_pad = jnp.pad(emb, ((0, NP - N), (0, 0)))

    tc0_out = _tc0(emb_pad, W1, as1, ad1)
    x_tabs, ast1, adt1 = tc0_out[0:8], tc0_out[8], tc0_out[9]
    e1, dpart1 = _sc_a(src2d, dst2d, ast1, adt1)
    dinv1 = _tc_recip(dpart1)
    att1 = _sc_a2(dst2d, e1, dinv1)
    outb1 = _sc_b_l1(src2d, dst2d, att1, *x_tabs)

    tc2_out = _tc2([outb1[u] for u in range(8)], b1, W2, as2, ad2)
    y_tabs, ast2, adt2 = tc2_out[0:4], tc2_out[4], tc2_out[5]
    e2, dpart2 = _sc_a(src2d, dst2d, ast2, adt2)
    dinv2 = _tc_recip(dpart2)
    att2 = _sc_a2(dst2d, e2, dinv2)
    outb2 = _sc_b_l2(src2d, dst2d, att2, *y_tabs)

    u2d = user_indices.reshape(NW, SUB)
    i2d = item_indices.reshape(NW, SUB)
    g = _sc_g(outb2[0], outb2[1], outb2[2], outb2[3], u2d, i2d)
    return _mlp_head(g[0:4], g[4:8], b2, fw1, fb1, fg, fbe, fw2, fb2,
                     pw1, pb1, pg, pbe, pw2, pb2, pw3, pb3)
